# baseline (speedup 1.0000x reference)
import functools
import math

import jax
import jax.numpy as jnp
from jax import lax
from jax.experimental import pallas as pl
from jax.experimental.pallas import tpu as pltpu
from jax.experimental.pallas import tpu_sc as plsc

B = 4
N0 = 1024
EMBED_DIM = 72
NUM_STAGES = 4
K_NEIGHBORS = 90
ALPHA = 1000.0
BETA = 100.0
G_BLK = 8
LN_ALPHA = math.log(ALPHA)
HALF_PI = math.pi / 2.0


def _fps_kernel(xyz_ref, out_ref, *, n, k):
    R = n // 128
    x = xyz_ref[0]
    y = xyz_ref[1]
    z = xyz_ref[2]
    iota = jax.lax.broadcasted_iota(jnp.int32, (R, 128), 0) * 128 + (
        jax.lax.broadcasted_iota(jnp.int32, (R, 128), 1)
    )
    d0 = (x - x[0, 0]) ** 2 + (y - y[0, 0]) ** 2 + (z - z[0, 0]) ** 2
    out_iota = jax.lax.broadcasted_iota(jnp.int32, (1, k), 1)

    def step(t, carry):
        min_d, out = carry
        m = jnp.max(min_d, axis=(0, 1), keepdims=True)
        idx = jnp.min(
            jnp.where(min_d == m, iota, n), axis=(0, 1), keepdims=True
        )
        sel = iota == idx
        px = jnp.sum(jnp.where(sel, x, 0.0), axis=(0, 1), keepdims=True)
        py = jnp.sum(jnp.where(sel, y, 0.0), axis=(0, 1), keepdims=True)
        pz = jnp.sum(jnp.where(sel, z, 0.0), axis=(0, 1), keepdims=True)
        d = (x - px) ** 2 + (y - py) ** 2 + (z - pz) ** 2
        out = jnp.where(out_iota == t, idx, out)
        return jnp.minimum(min_d, d), out

    out0 = jnp.zeros((1, k), jnp.int32)
    _, out = jax.lax.fori_loop(1, k, step, (d0, out0))
    out_ref[...] = out


def _fps_flat(pts_t, k):
    _, n = pts_t.shape
    out = pl.pallas_call(
        functools.partial(_fps_kernel, n=n, k=k),
        out_shape=jax.ShapeDtypeStruct((1, k), jnp.int32),
    )(pts_t.reshape(3, n // 128, 128))
    return out.reshape(k)


def _sc_gather_rows(tbl, idx):
    n_rows, d = idx.shape[0], tbl.shape[1]
    nw = 32
    per_w = n_rows // nw
    assert per_w * nw == n_rows
    if per_w <= 128:
        chunk = per_w
    elif 2 * 120 * d * 4 < 450_000 and per_w % 120 == 0:
        chunk = 120
    else:
        chunk = 72
    assert chunk <= 128 and chunk % 8 == 0 and per_w % chunk == 0
    n_chunks = per_w // chunk
    mesh = plsc.VectorSubcoreMesh(core_axis_name="c", subcore_axis_name="s")

    if n_chunks == 1:

        @functools.partial(
            pl.kernel,
            mesh=mesh,
            out_type=jax.ShapeDtypeStruct((n_rows, d), jnp.float32),
            scratch_types=[
                pltpu.VMEM((chunk,), jnp.int32),
                pltpu.VMEM((chunk, d), jnp.float32),
                pltpu.SemaphoreType.DMA,
            ],
            compiler_params=pltpu.CompilerParams(use_tc_tiling_on_sc=False),
        )
        def k1(tbl_hbm, idx_hbm, out_hbm, idx_v, rows_v, sem):
            wid = lax.axis_index("s") * 2 + lax.axis_index("c")
            base = wid * per_w
            pltpu.sync_copy(idx_hbm.at[pl.ds(base, chunk)], idx_v)
            pltpu.async_copy(tbl_hbm.at[idx_v], rows_v, sem).wait()
            pltpu.sync_copy(rows_v, out_hbm.at[pl.ds(base, chunk)])

        return k1(tbl, idx)

    assert n_chunks % 2 == 0
    n2 = n_chunks // 2

    @functools.partial(
        pl.kernel,
        mesh=mesh,
        out_type=jax.ShapeDtypeStruct((n_rows, d), jnp.float32),
        scratch_types=[
            pltpu.VMEM((per_w,), jnp.int32),
            pltpu.VMEM((chunk, d), jnp.float32),
            pltpu.VMEM((chunk, d), jnp.float32),
            pltpu.SemaphoreType.DMA,
            pltpu.SemaphoreType.DMA,
        ],
        compiler_params=pltpu.CompilerParams(use_tc_tiling_on_sc=False),
    )
    def k2(tbl_hbm, idx_hbm, out_hbm, idx_v, buf_a, buf_b, sem_a, sem_b):
        wid = lax.axis_index("s") * 2 + lax.axis_index("c")
        base_w = wid * per_w
        pltpu.sync_copy(idx_hbm.at[pl.ds(base_w, per_w)], idx_v)

        def start(j, buf, sem):
            return pltpu.async_copy(
                tbl_hbm.at[idx_v.at[pl.ds(j * chunk, chunk)]], buf, sem
            )

        start(0, buf_a, sem_a)

        def body(i, carry):
            start(2 * i + 1, buf_b, sem_b)
            pltpu.make_async_copy(tbl_hbm.at[pl.ds(0, chunk)], buf_a, sem_a).wait()
            pltpu.sync_copy(buf_a, out_hbm.at[pl.ds(base_w + 2 * i * chunk, chunk)])

            @pl.when(i < n2 - 1)
            def _():
                start(2 * i + 2, buf_a, sem_a)

            pltpu.make_async_copy(tbl_hbm.at[pl.ds(0, chunk)], buf_b, sem_b).wait()
            pltpu.sync_copy(
                buf_b, out_hbm.at[pl.ds(base_w + (2 * i + 1) * chunk, chunk)]
            )
            return carry

        lax.fori_loop(0, n2, body, 0)

    return k2(tbl, idx)


def _topk_kernel(lc_ref, xyzt_ref, out_ref, *, n, g, k, c):
    boff = pl.program_id(0) * n
    lc3 = lc_ref[0][:, c : c + 3]
    xt = xyzt_ref[0]
    ss_lc = jnp.sum(lc3 * lc3, axis=1, keepdims=True)
    ss_x = jnp.sum(xt * xt, axis=0, keepdims=True)
    dist = (
        ss_lc
        - 2.0 * jnp.dot(lc3, xt, preferred_element_type=jnp.float32)
        + ss_x
    )
    iota_n = jax.lax.broadcasted_iota(jnp.int32, (g, n), 1)
    lane_iota = jax.lax.broadcasted_iota(jnp.int32, (g, 128), 1)

    def step(j, carry):
        d, acc = carry
        m = jnp.min(d, axis=1, keepdims=True)
        idx = jnp.min(jnp.where(d == m, iota_n, n), axis=1, keepdims=True)
        d = jnp.where(iota_n == idx, jnp.inf, d)
        acc = jnp.where(lane_iota == j, idx + boff, acc)
        return d, acc

    acc0 = jnp.zeros((g, 128), jnp.int32)
    _, acc = jax.lax.fori_loop(0, k, step, (dist, acc0))
    out_ref[0] = acc


def _topk(lc_rows, xyz_t, n, g, k, c):
    dp = lc_rows.shape[-1]
    return pl.pallas_call(
        functools.partial(_topk_kernel, n=n, g=g, k=k, c=c),
        grid=(B,),
        in_specs=[
            pl.BlockSpec((1, g, dp), lambda b: (b, 0, 0)),
            pl.BlockSpec((1, 3, n), lambda b: (b, 0, 0)),
        ],
        out_specs=pl.BlockSpec((1, g, 128), lambda b: (b, 0, 0)),
        out_shape=jax.ShapeDtypeStruct((B, g, 128), jnp.int32),
    )(lc_rows, xyz_t)


def _stats_kernel(rows_ref, lc_ref, out_ref, *, c, kk, dp, gb):
    @pl.when(pl.program_id(0) == 0)
    def _init():
        out_ref[...] = jnp.zeros_like(out_ref)

    r3 = rows_ref[...]
    lc = lc_ref[...]
    s1 = jnp.sum(r3, axis=1)
    s2 = jnp.sum(r3 * r3, axis=1)
    t_sum = s1 - kk * lc
    t_sq = s2 - 2.0 * lc * s1 + kk * lc * lc
    lane = jax.lax.broadcasted_iota(jnp.int32, (gb, dp), 1)
    x_m = lane < c
    xyz_m = (lane >= c) & (lane < c + 3)
    vals = jnp.stack(
        [
            jnp.sum(jnp.where(x_m, t_sum, 0.0)),
            jnp.sum(jnp.where(x_m, t_sq, 0.0)),
            jnp.sum(jnp.where(xyz_m, t_sum, 0.0)),
            jnp.sum(jnp.where(xyz_m, t_sq, 0.0)),
        ]
    )
    lane4 = jax.lax.broadcasted_iota(jnp.int32, (1, 128), 1)
    row = (
        jnp.where(lane4 == 0, vals[0], 0.0)
        + jnp.where(lane4 == 1, vals[1], 0.0)
        + jnp.where(lane4 == 2, vals[2], 0.0)
        + jnp.where(lane4 == 3, vals[3], 0.0)
    )
    out_ref[...] += row


def _stats(rows3, lc_rows, c, kk):
    dp = rows3.shape[-1]
    gb = 32
    n_blocks = lc_rows.shape[0] // gb
    return pl.pallas_call(
        functools.partial(_stats_kernel, c=c, kk=kk, dp=dp, gb=gb),
        grid=(n_blocks,),
        in_specs=[
            pl.BlockSpec((gb, kk, dp), lambda i: (i, 0, 0)),
            pl.BlockSpec((gb, dp), lambda i: (i, 0)),
        ],
        out_specs=pl.BlockSpec((1, 128), lambda i: (0, 0)),
        out_shape=jax.ShapeDtypeStruct((1, 128), jnp.float32),
    )(rows3, lc_rows)


def _fused_kernel(rows_ref, lc_ref, st_ref, out_ref, *, c, fd, kk, dp, n_x, n_xyz, gb):
    c2 = 6 * fd
    st = st_ref[...]
    sum_x, sq_x = st[0, 0], st[0, 1]
    sum_xyz, sq_xyz = st[0, 2], st[0, 3]
    var_x = (sq_x - sum_x * sum_x / n_x) / (n_x - 1)
    var_xyz = (sq_xyz - sum_xyz * sum_xyz / n_xyz) / (n_xyz - 1)
    inv_x = 1.0 / (jnp.sqrt(var_x) + 1e-05)
    inv_xyz = 1.0 / (jnp.sqrt(var_xyz) + 1e-05)

    r3 = rows_ref[...]
    lc = lc_ref[...][:, None, :]
    xyz_n = (r3[..., c : c + 3] - lc[..., c : c + 3]) * inv_xyz
    x_n = (r3[..., 0:c] - lc[..., 0:c]) * inv_x

    li = jax.lax.broadcasted_iota(jnp.int32, (1, 1, c2), 2)
    f = (li % (2 * fd)) // 2
    scale = BETA * jnp.exp(f.astype(jnp.float32) * (-LN_ALPHA / fd))
    phase = jnp.where(li % 2 == 1, HALF_PI, 0.0)

    def bc(a):
        return jnp.broadcast_to(a, (gb, kk, 2 * fd))

    arg = jnp.concatenate(
        [bc(xyz_n[..., 0:1]), bc(xyz_n[..., 1:2]), bc(xyz_n[..., 2:3])], axis=-1
    )
    pe = jnp.sin(arg * scale + phase)
    lcx = jnp.broadcast_to(lc[..., 0:c], (gb, kk, c))
    feat = jnp.concatenate([x_n, lcx], axis=-1)
    w = (feat + pe) * pe
    out_ref[...] = jnp.max(w, axis=1) + jnp.sum(w, axis=1) * (1.0 / kk)


def _fused_pe(rows3, lc_rows, stats, c, fd, kk):
    dp = rows3.shape[-1]
    n_rows = lc_rows.shape[0]
    gb = 16 if c < 512 else 8
    n_blocks = n_rows // gb
    c2 = 6 * fd
    n_x = n_rows * kk * c
    n_xyz = n_rows * kk * 3
    return pl.pallas_call(
        functools.partial(
            _fused_kernel, c=c, fd=fd, kk=kk, dp=dp, n_x=n_x, n_xyz=n_xyz, gb=gb
        ),
        grid=(n_blocks,),
        in_specs=[
            pl.BlockSpec((gb, kk, dp), lambda i: (i, 0, 0)),
            pl.BlockSpec((gb, dp), lambda i: (i, 0)),
            pl.BlockSpec((1, 128), lambda i: (0, 0)),
        ],
        out_specs=pl.BlockSpec((gb, c2), lambda i: (i, 0)),
        out_shape=jax.ShapeDtypeStruct((n_rows, c2), jnp.float32),
    )(rows3, lc_rows, stats)


def _bn_tbl_kernel(x_ref, xyz_ref, w_ref, b_ref, o_ref, *, pad):
    x = x_ref[...]
    mean = jnp.mean(x, axis=0, keepdims=True)
    var = jnp.mean((x - mean) ** 2, axis=0, keepdims=True)
    xn = (x - mean) / jnp.sqrt(var + 1e-05) * w_ref[...] + b_ref[...]
    gel = 0.5 * xn * (1.0 + lax.erf(xn / jnp.sqrt(jnp.float32(2.0))))
    r = x.shape[0]
    o_ref[...] = jnp.concatenate(
        [gel, xyz_ref[...], jnp.zeros((r, pad), jnp.float32)], axis=-1
    )


def _bn_tbl(pooled, xyz3, w, b, dp_next):
    r, c2 = pooled.shape
    return pl.pallas_call(
        functools.partial(_bn_tbl_kernel, pad=dp_next - c2 - 3),
        out_shape=jax.ShapeDtypeStruct((r, dp_next), jnp.float32),
    )(pooled, xyz3, w.reshape(1, c2), b.reshape(1, c2))


def _bn_final_kernel(x_ref, w_ref, b_ref, o_ref, *, g):
    x = x_ref[...]
    mean = jnp.mean(x, axis=0, keepdims=True)
    var = jnp.mean((x - mean) ** 2, axis=0, keepdims=True)
    xn = (x - mean) / jnp.sqrt(var + 1e-05) * w_ref[...] + b_ref[...]
    gel = 0.5 * xn * (1.0 + lax.erf(xn / jnp.sqrt(jnp.float32(2.0))))
    c2 = gel.shape[-1]
    gel3 = gel.reshape(B, g, c2)
    o_ref[...] = jnp.max(gel3, axis=1) + jnp.sum(gel3, axis=1) * (1.0 / g)


def _bn_final(pooled, w, b, g):
    r, c2 = pooled.shape
    return pl.pallas_call(
        functools.partial(_bn_final_kernel, g=g),
        out_shape=jax.ShapeDtypeStruct((B, c2), jnp.float32),
    )(pooled, w.reshape(1, c2), b.reshape(1, c2))


def _pose_tbl_kernel(x_ref, xyz_ref, o_ref, *, fd, n, pad):
    v = x_ref[0]
    c2 = 6 * fd
    li = jax.lax.broadcasted_iota(jnp.int32, (1, c2), 1)
    f = (li % (2 * fd)) // 2
    scale = BETA * jnp.exp(f.astype(jnp.float32) * (-LN_ALPHA / fd))
    phase = jnp.where(li % 2 == 1, HALF_PI, 0.0)
    arg = jnp.concatenate(
        [
            jnp.broadcast_to(v[:, 0:1], (n, 2 * fd)),
            jnp.broadcast_to(v[:, 1:2], (n, 2 * fd)),
            jnp.broadcast_to(v[:, 2:3], (n, 2 * fd)),
        ],
        axis=-1,
    )
    o_ref[0] = jnp.concatenate(
        [
            jnp.sin(arg * scale + phase),
            xyz_ref[0],
            jnp.zeros((n, pad), jnp.float32),
        ],
        axis=-1,
    )


def _pose_tbl(x_t, xyz, fd, dp):
    _, n, _ = x_t.shape
    c2 = 6 * fd
    return pl.pallas_call(
        functools.partial(_pose_tbl_kernel, fd=fd, n=n, pad=dp - c2 - 3),
        grid=(B,),
        in_specs=[
            pl.BlockSpec((1, n, 3), lambda b: (b, 0, 0)),
            pl.BlockSpec((1, n, 3), lambda b: (b, 0, 0)),
        ],
        out_specs=pl.BlockSpec((1, n, dp), lambda b: (b, 0, 0)),
        out_shape=jax.ShapeDtypeStruct((B, n, dp), jnp.float32),
    )(x_t, xyz)


def kernel(xyz, x, bn_w0, bn_b0, bn_w1, bn_b1, bn_w2, bn_b2, bn_w3, bn_b3):
    bn = [(bn_w0, bn_b0), (bn_w1, bn_b1), (bn_w2, bn_b2), (bn_w3, bn_b3)]
    dp0 = ((3 + EMBED_DIM + 15) // 16) * 16
    tbl = _pose_tbl(x.transpose(0, 2, 1), xyz, EMBED_DIM // 6, dp0).reshape(
        B * N0, dp0
    )
    cur_xyz = xyz
    c = EMBED_DIM
    for i in range(NUM_STAGES):
        n = N0 >> i
        g = n // 2
        c2 = 2 * c
        fd = c2 // 6
        dp = ((3 + c + 15) // 16) * 16
        fps_idx = _fps_flat(cur_xyz.reshape(B * n, 3).T, g)
        lc_idx = jnp.minimum(
            fps_idx[None, :] + (jnp.arange(B, dtype=jnp.int32) * n)[:, None],
            B * n - 1,
        ).reshape(-1)
        lc_rows = _sc_gather_rows(tbl, lc_idx)
        flat_idx = _topk(
            lc_rows.reshape(B, g, dp),
            cur_xyz.transpose(0, 2, 1),
            n,
            g,
            K_NEIGHBORS,
            c,
        )[..., :K_NEIGHBORS].reshape(-1)
        rows3 = _sc_gather_rows(tbl, flat_idx).reshape(B * g, K_NEIGHBORS, dp)
        st = _stats(rows3, lc_rows, c, K_NEIGHBORS)
        pooled = _fused_pe(rows3, lc_rows, st, c, fd, K_NEIGHBORS)
        if i < NUM_STAGES - 1:
            xyz3 = lc_rows[:, c : c + 3]
            dp_next = ((3 + c2 + 15) // 16) * 16
            tbl = _bn_tbl(pooled, xyz3, bn[i][0], bn[i][1], dp_next)
            cur_xyz = xyz3.reshape(B, g, 3)
            c = c2
        else:
            return _bn_final(pooled, bn[i][0], bn[i][1], g)


# speedup vs baseline: 1.0074x; 1.0074x over previous
import functools
import math

import jax
import jax.numpy as jnp
from jax import lax
from jax.experimental import pallas as pl
from jax.experimental.pallas import tpu as pltpu
from jax.experimental.pallas import tpu_sc as plsc

B = 4
N0 = 1024
EMBED_DIM = 72
NUM_STAGES = 4
K_NEIGHBORS = 90
ALPHA = 1000.0
BETA = 100.0
G_BLK = 8
LN_ALPHA = math.log(ALPHA)
HALF_PI = math.pi / 2.0


def _fps_kernel(xyz_ref, out_ref, *, n, k):
    R = n // 128
    x = xyz_ref[0]
    y = xyz_ref[1]
    z = xyz_ref[2]
    iota = jax.lax.broadcasted_iota(jnp.int32, (R, 128), 0) * 128 + (
        jax.lax.broadcasted_iota(jnp.int32, (R, 128), 1)
    )
    d0 = (x - x[0, 0]) ** 2 + (y - y[0, 0]) ** 2 + (z - z[0, 0]) ** 2
    out_iota = jax.lax.broadcasted_iota(jnp.int32, (1, k), 1)

    def step(t, carry):
        min_d, out = carry
        m = jnp.max(min_d, axis=(0, 1), keepdims=True)
        idx = jnp.min(
            jnp.where(min_d == m, iota, n), axis=(0, 1), keepdims=True
        )
        sel = iota == idx
        px = jnp.sum(jnp.where(sel, x, 0.0), axis=(0, 1), keepdims=True)
        py = jnp.sum(jnp.where(sel, y, 0.0), axis=(0, 1), keepdims=True)
        pz = jnp.sum(jnp.where(sel, z, 0.0), axis=(0, 1), keepdims=True)
        d = (x - px) ** 2 + (y - py) ** 2 + (z - pz) ** 2
        out = jnp.where(out_iota == t, idx, out)
        return jnp.minimum(min_d, d), out

    out0 = jnp.zeros((1, k), jnp.int32)
    _, out = jax.lax.fori_loop(1, k, step, (d0, out0))
    out_ref[...] = out


def _fps_flat(pts_t, k):
    _, n = pts_t.shape
    out = pl.pallas_call(
        functools.partial(_fps_kernel, n=n, k=k),
        out_shape=jax.ShapeDtypeStruct((1, k), jnp.int32),
    )(pts_t.reshape(3, n // 128, 128))
    return out.reshape(k)


def _sc_gather_rows(tbl, idx):
    n_rows, d = idx.shape[0], tbl.shape[1]
    nw = 32
    per_w = n_rows // nw
    assert per_w * nw == n_rows
    if per_w <= 128:
        chunk = per_w
    elif 2 * 120 * d * 4 < 450_000 and per_w % 120 == 0:
        chunk = 120
    else:
        chunk = 72
    assert chunk <= 128 and chunk % 8 == 0 and per_w % chunk == 0
    n_chunks = per_w // chunk
    mesh = plsc.VectorSubcoreMesh(core_axis_name="c", subcore_axis_name="s")

    if n_chunks == 1:

        @functools.partial(
            pl.kernel,
            mesh=mesh,
            out_type=jax.ShapeDtypeStruct((n_rows, d), jnp.float32),
            scratch_types=[
                pltpu.VMEM((chunk,), jnp.int32),
                pltpu.VMEM((chunk, d), jnp.float32),
                pltpu.SemaphoreType.DMA,
            ],
            compiler_params=pltpu.CompilerParams(use_tc_tiling_on_sc=False),
        )
        def k1(tbl_hbm, idx_hbm, out_hbm, idx_v, rows_v, sem):
            wid = lax.axis_index("s") * 2 + lax.axis_index("c")
            base = wid * per_w
            pltpu.sync_copy(idx_hbm.at[pl.ds(base, chunk)], idx_v)
            pltpu.async_copy(tbl_hbm.at[idx_v], rows_v, sem).wait()
            pltpu.sync_copy(rows_v, out_hbm.at[pl.ds(base, chunk)])

        return k1(tbl, idx)

    assert n_chunks % 2 == 0
    n2 = n_chunks // 2

    @functools.partial(
        pl.kernel,
        mesh=mesh,
        out_type=jax.ShapeDtypeStruct((n_rows, d), jnp.float32),
        scratch_types=[
            pltpu.VMEM((per_w,), jnp.int32),
            pltpu.VMEM((chunk, d), jnp.float32),
            pltpu.VMEM((chunk, d), jnp.float32),
            pltpu.SemaphoreType.DMA,
            pltpu.SemaphoreType.DMA,
        ],
        compiler_params=pltpu.CompilerParams(use_tc_tiling_on_sc=False),
    )
    def k2(tbl_hbm, idx_hbm, out_hbm, idx_v, buf_a, buf_b, sem_a, sem_b):
        wid = lax.axis_index("s") * 2 + lax.axis_index("c")
        base_w = wid * per_w
        pltpu.sync_copy(idx_hbm.at[pl.ds(base_w, per_w)], idx_v)

        def start(j, buf, sem):
            return pltpu.async_copy(
                tbl_hbm.at[idx_v.at[pl.ds(j * chunk, chunk)]], buf, sem
            )

        start(0, buf_a, sem_a)

        def body(i, carry):
            start(2 * i + 1, buf_b, sem_b)
            pltpu.make_async_copy(tbl_hbm.at[pl.ds(0, chunk)], buf_a, sem_a).wait()
            pltpu.sync_copy(buf_a, out_hbm.at[pl.ds(base_w + 2 * i * chunk, chunk)])

            @pl.when(i < n2 - 1)
            def _():
                start(2 * i + 2, buf_a, sem_a)

            pltpu.make_async_copy(tbl_hbm.at[pl.ds(0, chunk)], buf_b, sem_b).wait()
            pltpu.sync_copy(
                buf_b, out_hbm.at[pl.ds(base_w + (2 * i + 1) * chunk, chunk)]
            )
            return carry

        lax.fori_loop(0, n2, body, 0)

    return k2(tbl, idx)


def _topk_kernel(lc_ref, xyzt_ref, out_ref, *, n, g, k):
    lc3 = lc_ref[0][:, :3]
    xt = xyzt_ref[0]
    ss_lc = jnp.sum(lc3 * lc3, axis=1, keepdims=True)
    ss_x = jnp.sum(xt * xt, axis=0, keepdims=True)
    dist = (
        ss_lc
        - 2.0 * jnp.dot(lc3, xt, preferred_element_type=jnp.float32)
        + ss_x
    )
    iota_n = jax.lax.broadcasted_iota(jnp.int32, (g, n), 1)
    lane_iota = jax.lax.broadcasted_iota(jnp.int32, (g, 128), 1)

    def step(j, carry):
        d, acc = carry
        m = jnp.min(d, axis=1, keepdims=True)
        idx = jnp.min(jnp.where(d == m, iota_n, n), axis=1, keepdims=True)
        d = jnp.where(iota_n == idx, jnp.inf, d)
        acc = jnp.where(lane_iota == j, idx, acc)
        return d, acc

    acc0 = jnp.zeros((g, 128), jnp.int32)
    _, acc = jax.lax.fori_loop(0, k, step, (dist, acc0))
    out_ref[0] = acc


def _topk(lc_rows, xyz_t, n, g, k):
    dp = lc_rows.shape[-1]
    return pl.pallas_call(
        functools.partial(_topk_kernel, n=n, g=g, k=k),
        grid=(B,),
        in_specs=[
            pl.BlockSpec((1, g, dp), lambda b: (b, 0, 0)),
            pl.BlockSpec((1, 3, n), lambda b: (b, 0, 0)),
        ],
        out_specs=pl.BlockSpec((1, g, 128), lambda b: (b, 0, 0)),
        out_shape=jax.ShapeDtypeStruct((B, g, 128), jnp.int32),
    )(lc_rows, xyz_t)


def _stats_kernel(rows_ref, lc_ref, out_ref, *, c, kk, dp, gb):
    @pl.when(pl.program_id(0) == 0)
    def _init():
        out_ref[...] = jnp.zeros_like(out_ref)

    r3 = rows_ref[...]
    lc = lc_ref[...]
    s1 = jnp.sum(r3, axis=1)
    s2 = jnp.sum(r3 * r3, axis=1)
    t_sum = s1 - kk * lc
    t_sq = s2 - 2.0 * lc * s1 + kk * lc * lc
    lane = jax.lax.broadcasted_iota(jnp.int32, (gb, dp), 1)
    xyz_m = lane < 3
    x_m = (lane >= 3) & (lane < 3 + c)
    vals = jnp.stack(
        [
            jnp.sum(jnp.where(x_m, t_sum, 0.0)),
            jnp.sum(jnp.where(x_m, t_sq, 0.0)),
            jnp.sum(jnp.where(xyz_m, t_sum, 0.0)),
            jnp.sum(jnp.where(xyz_m, t_sq, 0.0)),
        ]
    )
    lane4 = jax.lax.broadcasted_iota(jnp.int32, (1, 128), 1)
    row = (
        jnp.where(lane4 == 0, vals[0], 0.0)
        + jnp.where(lane4 == 1, vals[1], 0.0)
        + jnp.where(lane4 == 2, vals[2], 0.0)
        + jnp.where(lane4 == 3, vals[3], 0.0)
    )
    out_ref[...] += row


def _stats(rows3, lc_rows, c, kk):
    dp = rows3.shape[-1]
    gb = 32
    n_blocks = lc_rows.shape[0] // gb
    return pl.pallas_call(
        functools.partial(_stats_kernel, c=c, kk=kk, dp=dp, gb=gb),
        grid=(n_blocks,),
        in_specs=[
            pl.BlockSpec((gb, kk, dp), lambda i: (i, 0, 0)),
            pl.BlockSpec((gb, dp), lambda i: (i, 0)),
        ],
        out_specs=pl.BlockSpec((1, 128), lambda i: (0, 0)),
        out_shape=jax.ShapeDtypeStruct((1, 128), jnp.float32),
    )(rows3, lc_rows)


def _fused_kernel(rows_ref, lc_ref, st_ref, out_ref, *, c, fd, kk, dp, n_x, n_xyz, gb):
    c2 = 6 * fd
    st = st_ref[...]
    sum_x, sq_x = st[0, 0], st[0, 1]
    sum_xyz, sq_xyz = st[0, 2], st[0, 3]
    var_x = (sq_x - sum_x * sum_x / n_x) / (n_x - 1)
    var_xyz = (sq_xyz - sum_xyz * sum_xyz / n_xyz) / (n_xyz - 1)
    inv_x = 1.0 / (jnp.sqrt(var_x) + 1e-05)
    inv_xyz = 1.0 / (jnp.sqrt(var_xyz) + 1e-05)

    r3 = rows_ref[...]
    lc = lc_ref[...][:, None, :]
    xyz_n = (r3[..., 0:3] - lc[..., 0:3]) * inv_xyz
    x_n = (r3[..., 3 : 3 + c] - lc[..., 3 : 3 + c]) * inv_x

    li = jax.lax.broadcasted_iota(jnp.int32, (1, 1, c2), 2)
    f = (li % (2 * fd)) // 2
    scale = BETA * jnp.exp(f.astype(jnp.float32) * (-LN_ALPHA / fd))
    phase = jnp.where(li % 2 == 1, HALF_PI, 0.0)

    def bc(a):
        return jnp.broadcast_to(a, (gb, kk, 2 * fd))

    arg = jnp.concatenate(
        [bc(xyz_n[..., 0:1]), bc(xyz_n[..., 1:2]), bc(xyz_n[..., 2:3])], axis=-1
    )
    pe = jnp.sin(arg * scale + phase)
    lcx = jnp.broadcast_to(lc[..., 3 : 3 + c], (gb, kk, c))
    feat = jnp.concatenate([x_n, lcx], axis=-1)
    w = (feat + pe) * pe
    out_ref[...] = jnp.max(w, axis=1) + jnp.sum(w, axis=1) * (1.0 / kk)


def _fused_pe(rows3, lc_rows, stats, c, fd, kk):
    dp = rows3.shape[-1]
    n_rows = lc_rows.shape[0]
    gb = 16 if c < 512 else 8
    n_blocks = n_rows // gb
    c2 = 6 * fd
    n_x = n_rows * kk * c
    n_xyz = n_rows * kk * 3
    return pl.pallas_call(
        functools.partial(
            _fused_kernel, c=c, fd=fd, kk=kk, dp=dp, n_x=n_x, n_xyz=n_xyz, gb=gb
        ),
        grid=(n_blocks,),
        in_specs=[
            pl.BlockSpec((gb, kk, dp), lambda i: (i, 0, 0)),
            pl.BlockSpec((gb, dp), lambda i: (i, 0)),
            pl.BlockSpec((1, 128), lambda i: (0, 0)),
        ],
        out_specs=pl.BlockSpec((gb, c2), lambda i: (i, 0)),
        out_shape=jax.ShapeDtypeStruct((n_rows, c2), jnp.float32),
    )(rows3, lc_rows, stats)


def _bn_gelu_kernel(x_ref, w_ref, b_ref, o_ref):
    x = x_ref[...]
    mean = jnp.mean(x, axis=0, keepdims=True)
    var = jnp.mean((x - mean) ** 2, axis=0, keepdims=True)
    xn = (x - mean) / jnp.sqrt(var + 1e-05) * w_ref[...] + b_ref[...]
    o_ref[...] = 0.5 * xn * (1.0 + lax.erf(xn / jnp.sqrt(jnp.float32(2.0))))


def _bn_gelu(pooled, w, b):
    r, c2 = pooled.shape
    return pl.pallas_call(
        _bn_gelu_kernel,
        out_shape=jax.ShapeDtypeStruct((r, c2), jnp.float32),
    )(pooled, w.reshape(1, c2), b.reshape(1, c2))


def _pose_init_kernel(x_ref, o_ref, *, fd, n):
    v = x_ref[0]
    c2 = 6 * fd
    li = jax.lax.broadcasted_iota(jnp.int32, (1, c2), 1)
    f = (li % (2 * fd)) // 2
    scale = BETA * jnp.exp(f.astype(jnp.float32) * (-LN_ALPHA / fd))
    phase = jnp.where(li % 2 == 1, HALF_PI, 0.0)
    arg = jnp.concatenate(
        [
            jnp.broadcast_to(v[:, 0:1], (n, 2 * fd)),
            jnp.broadcast_to(v[:, 1:2], (n, 2 * fd)),
            jnp.broadcast_to(v[:, 2:3], (n, 2 * fd)),
        ],
        axis=-1,
    )
    o_ref[0] = jnp.sin(arg * scale + phase)


def _pose_initial(x_t, fd):
    _, n, _ = x_t.shape
    c2 = 6 * fd
    return pl.pallas_call(
        functools.partial(_pose_init_kernel, fd=fd, n=n),
        grid=(B,),
        in_specs=[pl.BlockSpec((1, n, 3), lambda b: (b, 0, 0))],
        out_specs=pl.BlockSpec((1, n, c2), lambda b: (b, 0, 0)),
        out_shape=jax.ShapeDtypeStruct((B, n, c2), jnp.float32),
    )(x_t)


def _final_kernel(x_ref, o_ref, *, g):
    r = x_ref[...]
    o_ref[...] = jnp.max(r, axis=1) + jnp.sum(r, axis=1) * (1.0 / g)


def _final_reduce(x):
    _, g, c2 = x.shape
    return pl.pallas_call(
        functools.partial(_final_kernel, g=g),
        out_shape=jax.ShapeDtypeStruct((B, c2), jnp.float32),
    )(x)


def kernel(xyz, x, bn_w0, bn_b0, bn_w1, bn_b1, bn_w2, bn_b2, bn_w3, bn_b3):
    bn = [(bn_w0, bn_b0), (bn_w1, bn_b1), (bn_w2, bn_b2), (bn_w3, bn_b3)]
    xfeat = _pose_initial(x.transpose(0, 2, 1), EMBED_DIM // 6)
    cur_xyz = xyz
    for i in range(NUM_STAGES):
        n = N0 >> i
        g = n // 2
        c = xfeat.shape[-1]
        c2 = 2 * c
        fd = c2 // 6
        dp = ((3 + c + 15) // 16) * 16
        fps_idx = _fps_flat(cur_xyz.reshape(B * n, 3).T, g)
        tbl = jnp.concatenate(
            [cur_xyz, xfeat, jnp.zeros((B, n, dp - 3 - c), jnp.float32)], axis=-1
        ).reshape(B * n, dp)
        lc_idx = jnp.minimum(
            fps_idx[None, :] + (jnp.arange(B, dtype=jnp.int32) * n)[:, None],
            B * n - 1,
        ).reshape(-1)
        lc_rows = _sc_gather_rows(tbl, lc_idx)
        knn_idx = _topk(
            lc_rows.reshape(B, g, dp), cur_xyz.transpose(0, 2, 1), n, g, K_NEIGHBORS
        )[..., :K_NEIGHBORS]
        flat_idx = (
            knn_idx + (jnp.arange(B, dtype=jnp.int32) * n)[:, None, None]
        ).reshape(-1)
        rows3 = _sc_gather_rows(tbl, flat_idx).reshape(B * g, K_NEIGHBORS, dp)
        st = _stats(rows3, lc_rows, c, K_NEIGHBORS)
        pooled = _fused_pe(rows3, lc_rows, st, c, fd, K_NEIGHBORS)
        xfeat = _bn_gelu(pooled, bn[i][0], bn[i][1]).reshape(B, g, c2)
        cur_xyz = lc_rows[:, :3].reshape(B, g, 3)
    return _final_reduce(xfeat)


# speedup vs baseline: 1.0291x; 1.0215x over previous
import functools
import math

import jax
import jax.numpy as jnp
from jax import lax
from jax.experimental import pallas as pl
from jax.experimental.pallas import tpu as pltpu
from jax.experimental.pallas import tpu_sc as plsc

B = 4
N0 = 1024
EMBED_DIM = 72
NUM_STAGES = 4
K_NEIGHBORS = 90
ALPHA = 1000.0
BETA = 100.0
G_BLK = 8
LN_ALPHA = math.log(ALPHA)
HALF_PI = math.pi / 2.0


def _fps_kernel(xyz_ref, out_ref, *, n, k):
    R = n // 128
    x = xyz_ref[0]
    y = xyz_ref[1]
    z = xyz_ref[2]
    iota = jax.lax.broadcasted_iota(jnp.int32, (R, 128), 0) * 128 + (
        jax.lax.broadcasted_iota(jnp.int32, (R, 128), 1)
    )
    d0 = (x - x[0, 0]) ** 2 + (y - y[0, 0]) ** 2 + (z - z[0, 0]) ** 2
    out_iota = jax.lax.broadcasted_iota(jnp.int32, (1, k), 1)

    def step(t, carry):
        min_d, out = carry
        m = jnp.max(min_d, axis=(0, 1), keepdims=True)
        idx = jnp.min(
            jnp.where(min_d == m, iota, n), axis=(0, 1), keepdims=True
        )
        sel = iota == idx
        px = jnp.sum(jnp.where(sel, x, 0.0), axis=(0, 1), keepdims=True)
        py = jnp.sum(jnp.where(sel, y, 0.0), axis=(0, 1), keepdims=True)
        pz = jnp.sum(jnp.where(sel, z, 0.0), axis=(0, 1), keepdims=True)
        d = (x - px) ** 2 + (y - py) ** 2 + (z - pz) ** 2
        out = jnp.where(out_iota == t, idx, out)
        return jnp.minimum(min_d, d), out

    out0 = jnp.zeros((1, k), jnp.int32)
    _, out = jax.lax.fori_loop(1, k, step, (d0, out0))
    out_ref[...] = out


def _fps_flat(pts_t, k):
    _, n = pts_t.shape
    out = pl.pallas_call(
        functools.partial(_fps_kernel, n=n, k=k),
        out_shape=jax.ShapeDtypeStruct((1, k), jnp.int32),
    )(pts_t.reshape(3, n // 128, 128))
    return out.reshape(k)


def _sc_gather_rows(tbl, idx):
    n_rows, d = idx.shape[0], tbl.shape[1]
    nw = 32
    per_w = n_rows // nw
    assert per_w * nw == n_rows
    if per_w <= 128:
        chunk = per_w
    elif 2 * 120 * d * 4 < 450_000 and per_w % 120 == 0:
        chunk = 120
    else:
        chunk = 72
    assert chunk <= 128 and chunk % 8 == 0 and per_w % chunk == 0
    n_chunks = per_w // chunk
    mesh = plsc.VectorSubcoreMesh(core_axis_name="c", subcore_axis_name="s")

    if n_chunks == 1:

        @functools.partial(
            pl.kernel,
            mesh=mesh,
            out_type=jax.ShapeDtypeStruct((n_rows, d), jnp.float32),
            scratch_types=[
                pltpu.VMEM((chunk,), jnp.int32),
                pltpu.VMEM((chunk, d), jnp.float32),
                pltpu.SemaphoreType.DMA,
            ],
            compiler_params=pltpu.CompilerParams(use_tc_tiling_on_sc=False),
        )
        def k1(tbl_hbm, idx_hbm, out_hbm, idx_v, rows_v, sem):
            wid = lax.axis_index("s") * 2 + lax.axis_index("c")
            base = wid * per_w
            pltpu.sync_copy(idx_hbm.at[pl.ds(base, chunk)], idx_v)
            pltpu.async_copy(tbl_hbm.at[idx_v], rows_v, sem).wait()
            pltpu.sync_copy(rows_v, out_hbm.at[pl.ds(base, chunk)])

        return k1(tbl, idx)

    assert n_chunks % 2 == 0
    n2 = n_chunks // 2

    @functools.partial(
        pl.kernel,
        mesh=mesh,
        out_type=jax.ShapeDtypeStruct((n_rows, d), jnp.float32),
        scratch_types=[
            pltpu.VMEM((per_w,), jnp.int32),
            pltpu.VMEM((chunk, d), jnp.float32),
            pltpu.VMEM((chunk, d), jnp.float32),
            pltpu.SemaphoreType.DMA,
            pltpu.SemaphoreType.DMA,
        ],
        compiler_params=pltpu.CompilerParams(use_tc_tiling_on_sc=False),
    )
    def k2(tbl_hbm, idx_hbm, out_hbm, idx_v, buf_a, buf_b, sem_a, sem_b):
        wid = lax.axis_index("s") * 2 + lax.axis_index("c")
        base_w = wid * per_w
        pltpu.sync_copy(idx_hbm.at[pl.ds(base_w, per_w)], idx_v)

        def start(j, buf, sem):
            return pltpu.async_copy(
                tbl_hbm.at[idx_v.at[pl.ds(j * chunk, chunk)]], buf, sem
            )

        start(0, buf_a, sem_a)

        def body(i, carry):
            start(2 * i + 1, buf_b, sem_b)
            pltpu.make_async_copy(tbl_hbm.at[pl.ds(0, chunk)], buf_a, sem_a).wait()
            pltpu.sync_copy(buf_a, out_hbm.at[pl.ds(base_w + 2 * i * chunk, chunk)])

            @pl.when(i < n2 - 1)
            def _():
                start(2 * i + 2, buf_a, sem_a)

            pltpu.make_async_copy(tbl_hbm.at[pl.ds(0, chunk)], buf_b, sem_b).wait()
            pltpu.sync_copy(
                buf_b, out_hbm.at[pl.ds(base_w + (2 * i + 1) * chunk, chunk)]
            )
            return carry

        lax.fori_loop(0, n2, body, 0)

    return k2(tbl, idx)


def _topk_kernel(lc_ref, xyzt_ref, out_ref, *, n, g, k):
    lc3 = lc_ref[0][:, :3]
    xt = xyzt_ref[0]
    ss_lc = jnp.sum(lc3 * lc3, axis=1, keepdims=True)
    ss_x = jnp.sum(xt * xt, axis=0, keepdims=True)
    dist = (
        ss_lc
        - 2.0 * jnp.dot(lc3, xt, preferred_element_type=jnp.float32)
        + ss_x
    )
    iota_n = jax.lax.broadcasted_iota(jnp.int32, (g, n), 1)
    lane_iota = jax.lax.broadcasted_iota(jnp.int32, (g, 128), 1)

    def step(j, carry):
        d, acc = carry
        m = jnp.min(d, axis=1, keepdims=True)
        idx = jnp.min(jnp.where(d == m, iota_n, n), axis=1, keepdims=True)
        d = jnp.where(iota_n == idx, jnp.inf, d)
        acc = jnp.where(lane_iota == j, idx, acc)
        return d, acc

    acc0 = jnp.zeros((g, 128), jnp.int32)
    _, acc = jax.lax.fori_loop(0, k, step, (dist, acc0))
    out_ref[0] = acc


def _topk(lc_rows, xyz_t, n, g, k):
    dp = lc_rows.shape[-1]
    return pl.pallas_call(
        functools.partial(_topk_kernel, n=n, g=g, k=k),
        grid=(B,),
        in_specs=[
            pl.BlockSpec((1, g, dp), lambda b: (b, 0, 0)),
            pl.BlockSpec((1, 3, n), lambda b: (b, 0, 0)),
        ],
        out_specs=pl.BlockSpec((1, g, 128), lambda b: (b, 0, 0)),
        out_shape=jax.ShapeDtypeStruct((B, g, 128), jnp.int32),
    )(lc_rows, xyz_t)


def _stats_kernel(rows_ref, lc_ref, out_ref, *, c, kk, dp, gb):
    @pl.when(pl.program_id(0) == 0)
    def _init():
        out_ref[...] = jnp.zeros_like(out_ref)

    r3 = rows_ref[...]
    lc = lc_ref[...]
    s1 = jnp.sum(r3, axis=1)
    s2 = jnp.sum(r3 * r3, axis=1)
    t_sum = s1 - kk * lc
    t_sq = s2 - 2.0 * lc * s1 + kk * lc * lc
    lane = jax.lax.broadcasted_iota(jnp.int32, (gb, dp), 1)
    xyz_m = lane < 3
    x_m = (lane >= 3) & (lane < 3 + c)
    vals = jnp.stack(
        [
            jnp.sum(jnp.where(x_m, t_sum, 0.0)),
            jnp.sum(jnp.where(x_m, t_sq, 0.0)),
            jnp.sum(jnp.where(xyz_m, t_sum, 0.0)),
            jnp.sum(jnp.where(xyz_m, t_sq, 0.0)),
        ]
    )
    lane4 = jax.lax.broadcasted_iota(jnp.int32, (1, 128), 1)
    row = (
        jnp.where(lane4 == 0, vals[0], 0.0)
        + jnp.where(lane4 == 1, vals[1], 0.0)
        + jnp.where(lane4 == 2, vals[2], 0.0)
        + jnp.where(lane4 == 3, vals[3], 0.0)
    )
    out_ref[...] += row


def _stats(rows3, lc_rows, c, kk):
    dp = rows3.shape[-1]
    gb = 64
    n_blocks = lc_rows.shape[0] // gb
    return pl.pallas_call(
        functools.partial(_stats_kernel, c=c, kk=kk, dp=dp, gb=gb),
        grid=(n_blocks,),
        in_specs=[
            pl.BlockSpec((gb, kk, dp), lambda i: (i, 0, 0)),
            pl.BlockSpec((gb, dp), lambda i: (i, 0)),
        ],
        out_specs=pl.BlockSpec((1, 128), lambda i: (0, 0)),
        out_shape=jax.ShapeDtypeStruct((1, 128), jnp.float32),
    )(rows3, lc_rows)


def _fused_kernel(rows_ref, lc_ref, st_ref, out_ref, *, c, fd, kk, dp, n_x, n_xyz, gb):
    c2 = 6 * fd
    st = st_ref[...]
    sum_x, sq_x = st[0, 0], st[0, 1]
    sum_xyz, sq_xyz = st[0, 2], st[0, 3]
    var_x = (sq_x - sum_x * sum_x / n_x) / (n_x - 1)
    var_xyz = (sq_xyz - sum_xyz * sum_xyz / n_xyz) / (n_xyz - 1)
    inv_x = 1.0 / (jnp.sqrt(var_x) + 1e-05)
    inv_xyz = 1.0 / (jnp.sqrt(var_xyz) + 1e-05)

    r3 = rows_ref[...]
    lc = lc_ref[...][:, None, :]
    xyz_n = (r3[..., 0:3] - lc[..., 0:3]) * inv_xyz
    x_n = (r3[..., 3 : 3 + c] - lc[..., 3 : 3 + c]) * inv_x

    li = jax.lax.broadcasted_iota(jnp.int32, (1, 1, c2), 2)
    f = (li % (2 * fd)) // 2
    scale = BETA * jnp.exp(f.astype(jnp.float32) * (-LN_ALPHA / fd))
    phase = jnp.where(li % 2 == 1, HALF_PI, 0.0)

    def bc(a):
        return jnp.broadcast_to(a, (gb, kk, 2 * fd))

    arg = jnp.concatenate(
        [bc(xyz_n[..., 0:1]), bc(xyz_n[..., 1:2]), bc(xyz_n[..., 2:3])], axis=-1
    )
    pe = jnp.sin(arg * scale + phase)
    lcx = jnp.broadcast_to(lc[..., 3 : 3 + c], (gb, kk, c))
    feat = jnp.concatenate([x_n, lcx], axis=-1)
    w = (feat + pe) * pe
    out_ref[...] = jnp.max(w, axis=1) + jnp.sum(w, axis=1) * (1.0 / kk)


def _fused_pe(rows3, lc_rows, stats, c, fd, kk):
    dp = rows3.shape[-1]
    n_rows = lc_rows.shape[0]
    gb = 32 if c < 256 else (16 if c < 512 else 8)
    n_blocks = n_rows // gb
    c2 = 6 * fd
    n_x = n_rows * kk * c
    n_xyz = n_rows * kk * 3
    return pl.pallas_call(
        functools.partial(
            _fused_kernel, c=c, fd=fd, kk=kk, dp=dp, n_x=n_x, n_xyz=n_xyz, gb=gb
        ),
        grid=(n_blocks,),
        in_specs=[
            pl.BlockSpec((gb, kk, dp), lambda i: (i, 0, 0)),
            pl.BlockSpec((gb, dp), lambda i: (i, 0)),
            pl.BlockSpec((1, 128), lambda i: (0, 0)),
        ],
        out_specs=pl.BlockSpec((gb, c2), lambda i: (i, 0)),
        out_shape=jax.ShapeDtypeStruct((n_rows, c2), jnp.float32),
    )(rows3, lc_rows, stats)


def _bn_gelu_kernel(x_ref, w_ref, b_ref, o_ref):
    x = x_ref[...]
    mean = jnp.mean(x, axis=0, keepdims=True)
    var = jnp.mean((x - mean) ** 2, axis=0, keepdims=True)
    xn = (x - mean) / jnp.sqrt(var + 1e-05) * w_ref[...] + b_ref[...]
    o_ref[...] = 0.5 * xn * (1.0 + lax.erf(xn / jnp.sqrt(jnp.float32(2.0))))


def _bn_gelu(pooled, w, b):
    r, c2 = pooled.shape
    return pl.pallas_call(
        _bn_gelu_kernel,
        out_shape=jax.ShapeDtypeStruct((r, c2), jnp.float32),
    )(pooled, w.reshape(1, c2), b.reshape(1, c2))


def _pose_init_kernel(x_ref, o_ref, *, fd, n):
    v = x_ref[0]
    c2 = 6 * fd
    li = jax.lax.broadcasted_iota(jnp.int32, (1, c2), 1)
    f = (li % (2 * fd)) // 2
    scale = BETA * jnp.exp(f.astype(jnp.float32) * (-LN_ALPHA / fd))
    phase = jnp.where(li % 2 == 1, HALF_PI, 0.0)
    arg = jnp.concatenate(
        [
            jnp.broadcast_to(v[:, 0:1], (n, 2 * fd)),
            jnp.broadcast_to(v[:, 1:2], (n, 2 * fd)),
            jnp.broadcast_to(v[:, 2:3], (n, 2 * fd)),
        ],
        axis=-1,
    )
    o_ref[0] = jnp.sin(arg * scale + phase)


def _pose_initial(x_t, fd):
    _, n, _ = x_t.shape
    c2 = 6 * fd
    return pl.pallas_call(
        functools.partial(_pose_init_kernel, fd=fd, n=n),
        grid=(B,),
        in_specs=[pl.BlockSpec((1, n, 3), lambda b: (b, 0, 0))],
        out_specs=pl.BlockSpec((1, n, c2), lambda b: (b, 0, 0)),
        out_shape=jax.ShapeDtypeStruct((B, n, c2), jnp.float32),
    )(x_t)


def _final_kernel(x_ref, o_ref, *, g):
    r = x_ref[...]
    o_ref[...] = jnp.max(r, axis=1) + jnp.sum(r, axis=1) * (1.0 / g)


def _final_reduce(x):
    _, g, c2 = x.shape
    return pl.pallas_call(
        functools.partial(_final_kernel, g=g),
        out_shape=jax.ShapeDtypeStruct((B, c2), jnp.float32),
    )(x)


def kernel(xyz, x, bn_w0, bn_b0, bn_w1, bn_b1, bn_w2, bn_b2, bn_w3, bn_b3):
    bn = [(bn_w0, bn_b0), (bn_w1, bn_b1), (bn_w2, bn_b2), (bn_w3, bn_b3)]
    xfeat = _pose_initial(x.transpose(0, 2, 1), EMBED_DIM // 6)
    cur_xyz = xyz
    for i in range(NUM_STAGES):
        n = N0 >> i
        g = n // 2
        c = xfeat.shape[-1]
        c2 = 2 * c
        fd = c2 // 6
        dp = ((3 + c + 15) // 16) * 16
        fps_idx = _fps_flat(cur_xyz.reshape(B * n, 3).T, g)
        tbl = jnp.concatenate(
            [cur_xyz, xfeat, jnp.zeros((B, n, dp - 3 - c), jnp.float32)], axis=-1
        ).reshape(B * n, dp)
        lc_idx = jnp.minimum(
            fps_idx[None, :] + (jnp.arange(B, dtype=jnp.int32) * n)[:, None],
            B * n - 1,
        ).reshape(-1)
        lc_rows = _sc_gather_rows(tbl, lc_idx)
        knn_idx = _topk(
            lc_rows.reshape(B, g, dp), cur_xyz.transpose(0, 2, 1), n, g, K_NEIGHBORS
        )[..., :K_NEIGHBORS]
        flat_idx = (
            knn_idx + (jnp.arange(B, dtype=jnp.int32) * n)[:, None, None]
        ).reshape(-1)
        rows3 = _sc_gather_rows(tbl, flat_idx).reshape(B * g, K_NEIGHBORS, dp)
        st = _stats(rows3, lc_rows, c, K_NEIGHBORS)
        pooled = _fused_pe(rows3, lc_rows, st, c, fd, K_NEIGHBORS)
        xfeat = _bn_gelu(pooled, bn[i][0], bn[i][1]).reshape(B, g, c2)
        cur_xyz = lc_rows[:, :3].reshape(B, g, 3)
    return _final_reduce(xfeat)


# speedup vs baseline: 1.0610x; 1.0310x over previous
import functools
import math

import jax
import jax.numpy as jnp
from jax import lax
from jax.experimental import pallas as pl
from jax.experimental.pallas import tpu as pltpu
from jax.experimental.pallas import tpu_sc as plsc

B = 4
N0 = 1024
EMBED_DIM = 72
NUM_STAGES = 4
K_NEIGHBORS = 90
ALPHA = 1000.0
BETA = 100.0
G_BLK = 8
LN_ALPHA = math.log(ALPHA)
HALF_PI = math.pi / 2.0


def _fps_kernel(xyz_ref, out_ref, *, n, k):
    R = n // 128
    x = xyz_ref[0]
    y = xyz_ref[1]
    z = xyz_ref[2]
    iota = jax.lax.broadcasted_iota(jnp.int32, (R, 128), 0) * 128 + (
        jax.lax.broadcasted_iota(jnp.int32, (R, 128), 1)
    )
    d0 = (x - x[0, 0]) ** 2 + (y - y[0, 0]) ** 2 + (z - z[0, 0]) ** 2
    out_iota = jax.lax.broadcasted_iota(jnp.int32, (1, k), 1)

    def step(t, carry):
        min_d, out = carry
        m = jnp.max(min_d, axis=(0, 1), keepdims=True)
        idx = jnp.min(
            jnp.where(min_d == m, iota, n), axis=(0, 1), keepdims=True
        )
        sel = iota == idx
        px = jnp.sum(jnp.where(sel, x, 0.0), axis=(0, 1), keepdims=True)
        py = jnp.sum(jnp.where(sel, y, 0.0), axis=(0, 1), keepdims=True)
        pz = jnp.sum(jnp.where(sel, z, 0.0), axis=(0, 1), keepdims=True)
        d = (x - px) ** 2 + (y - py) ** 2 + (z - pz) ** 2
        out = jnp.where(out_iota == t, idx, out)
        return jnp.minimum(min_d, d), out

    out0 = jnp.zeros((1, k), jnp.int32)
    _, out = jax.lax.fori_loop(1, k, step, (d0, out0))
    out_ref[...] = out


def _fps_flat(pts_t, k):
    _, n = pts_t.shape
    out = pl.pallas_call(
        functools.partial(_fps_kernel, n=n, k=k),
        out_shape=jax.ShapeDtypeStruct((1, k), jnp.int32),
    )(pts_t.reshape(3, n // 128, 128))
    return out.reshape(k)


def _sc_gather_rows(tbl, idx):
    n_rows, d = idx.shape[0], tbl.shape[1]
    nw = 32
    per_w = n_rows // nw
    assert per_w * nw == n_rows
    if per_w <= 128:
        chunk = per_w
    elif 2 * 120 * d * 4 < 450_000 and per_w % 120 == 0:
        chunk = 120
    else:
        chunk = 72
    assert chunk <= 128 and chunk % 8 == 0 and per_w % chunk == 0
    n_chunks = per_w // chunk
    mesh = plsc.VectorSubcoreMesh(core_axis_name="c", subcore_axis_name="s")

    if n_chunks == 1:

        @functools.partial(
            pl.kernel,
            mesh=mesh,
            out_type=jax.ShapeDtypeStruct((n_rows, d), jnp.float32),
            scratch_types=[
                pltpu.VMEM((chunk,), jnp.int32),
                pltpu.VMEM((chunk, d), jnp.float32),
                pltpu.SemaphoreType.DMA,
            ],
            compiler_params=pltpu.CompilerParams(use_tc_tiling_on_sc=False),
        )
        def k1(tbl_hbm, idx_hbm, out_hbm, idx_v, rows_v, sem):
            wid = lax.axis_index("s") * 2 + lax.axis_index("c")
            base = wid * per_w
            pltpu.sync_copy(idx_hbm.at[pl.ds(base, chunk)], idx_v)
            pltpu.async_copy(tbl_hbm.at[idx_v], rows_v, sem).wait()
            pltpu.sync_copy(rows_v, out_hbm.at[pl.ds(base, chunk)])

        return k1(tbl, idx)

    assert n_chunks % 2 == 0
    n2 = n_chunks // 2

    @functools.partial(
        pl.kernel,
        mesh=mesh,
        out_type=jax.ShapeDtypeStruct((n_rows, d), jnp.float32),
        scratch_types=[
            pltpu.VMEM((per_w,), jnp.int32),
            pltpu.VMEM((chunk, d), jnp.float32),
            pltpu.VMEM((chunk, d), jnp.float32),
            pltpu.SemaphoreType.DMA,
            pltpu.SemaphoreType.DMA,
        ],
        compiler_params=pltpu.CompilerParams(use_tc_tiling_on_sc=False),
    )
    def k2(tbl_hbm, idx_hbm, out_hbm, idx_v, buf_a, buf_b, sem_a, sem_b):
        wid = lax.axis_index("s") * 2 + lax.axis_index("c")
        base_w = wid * per_w
        pltpu.sync_copy(idx_hbm.at[pl.ds(base_w, per_w)], idx_v)

        def start(j, buf, sem):
            return pltpu.async_copy(
                tbl_hbm.at[idx_v.at[pl.ds(j * chunk, chunk)]], buf, sem
            )

        start(0, buf_a, sem_a)

        def body(i, carry):
            start(2 * i + 1, buf_b, sem_b)
            pltpu.make_async_copy(tbl_hbm.at[pl.ds(0, chunk)], buf_a, sem_a).wait()
            pltpu.sync_copy(buf_a, out_hbm.at[pl.ds(base_w + 2 * i * chunk, chunk)])

            @pl.when(i < n2 - 1)
            def _():
                start(2 * i + 2, buf_a, sem_a)

            pltpu.make_async_copy(tbl_hbm.at[pl.ds(0, chunk)], buf_b, sem_b).wait()
            pltpu.sync_copy(
                buf_b, out_hbm.at[pl.ds(base_w + (2 * i + 1) * chunk, chunk)]
            )
            return carry

        lax.fori_loop(0, n2, body, 0)

    return k2(tbl, idx)


def _topk_kernel(lc_ref, xyzt_ref, out_ref, *, n, g, k):
    lc3 = lc_ref[0][:, :3]
    xt = xyzt_ref[0]
    ss_lc = jnp.sum(lc3 * lc3, axis=1, keepdims=True)
    ss_x = jnp.sum(xt * xt, axis=0, keepdims=True)
    dist = (
        ss_lc
        - 2.0 * jnp.dot(lc3, xt, preferred_element_type=jnp.float32)
        + ss_x
    )
    iota_n = jax.lax.broadcasted_iota(jnp.int32, (g, n), 1)
    lane_iota = jax.lax.broadcasted_iota(jnp.int32, (g, 128), 1)
    bits = jax.lax.bitcast_convert_type(jnp.maximum(dist, 0.0), jnp.int32)
    comb = jnp.bitwise_or(jnp.bitwise_and(bits, jnp.int32(-1024)), iota_n)

    def step(j, carry):
        cb, acc = carry
        m = jnp.min(cb, axis=1, keepdims=True)
        acc = jnp.where(lane_iota == j, jnp.bitwise_and(m, 1023), acc)
        cb = jnp.where(cb == m, jnp.int32(0x7FFFFFFF), cb)
        return cb, acc

    acc0 = jnp.zeros((g, 128), jnp.int32)
    _, acc = jax.lax.fori_loop(0, k, step, (comb, acc0))
    out_ref[0] = acc


def _topk(lc_rows, xyz_t, n, g, k):
    dp = lc_rows.shape[-1]
    return pl.pallas_call(
        functools.partial(_topk_kernel, n=n, g=g, k=k),
        grid=(B,),
        in_specs=[
            pl.BlockSpec((1, g, dp), lambda b: (b, 0, 0)),
            pl.BlockSpec((1, 3, n), lambda b: (b, 0, 0)),
        ],
        out_specs=pl.BlockSpec((1, g, 128), lambda b: (b, 0, 0)),
        out_shape=jax.ShapeDtypeStruct((B, g, 128), jnp.int32),
    )(lc_rows, xyz_t)


def _stats_kernel(rows_ref, lc_ref, out_ref, *, c, kk, dp, gb):
    @pl.when(pl.program_id(0) == 0)
    def _init():
        out_ref[...] = jnp.zeros_like(out_ref)

    r3 = rows_ref[...]
    lc = lc_ref[...]
    s1 = jnp.sum(r3, axis=1)
    s2 = jnp.sum(r3 * r3, axis=1)
    t_sum = s1 - kk * lc
    t_sq = s2 - 2.0 * lc * s1 + kk * lc * lc
    lane = jax.lax.broadcasted_iota(jnp.int32, (gb, dp), 1)
    xyz_m = lane < 3
    x_m = (lane >= 3) & (lane < 3 + c)
    vals = jnp.stack(
        [
            jnp.sum(jnp.where(x_m, t_sum, 0.0)),
            jnp.sum(jnp.where(x_m, t_sq, 0.0)),
            jnp.sum(jnp.where(xyz_m, t_sum, 0.0)),
            jnp.sum(jnp.where(xyz_m, t_sq, 0.0)),
        ]
    )
    lane4 = jax.lax.broadcasted_iota(jnp.int32, (1, 128), 1)
    row = (
        jnp.where(lane4 == 0, vals[0], 0.0)
        + jnp.where(lane4 == 1, vals[1], 0.0)
        + jnp.where(lane4 == 2, vals[2], 0.0)
        + jnp.where(lane4 == 3, vals[3], 0.0)
    )
    out_ref[...] += row


def _stats(rows3, lc_rows, c, kk):
    dp = rows3.shape[-1]
    gb = 64
    n_blocks = lc_rows.shape[0] // gb
    return pl.pallas_call(
        functools.partial(_stats_kernel, c=c, kk=kk, dp=dp, gb=gb),
        grid=(n_blocks,),
        in_specs=[
            pl.BlockSpec((gb, kk, dp), lambda i: (i, 0, 0)),
            pl.BlockSpec((gb, dp), lambda i: (i, 0)),
        ],
        out_specs=pl.BlockSpec((1, 128), lambda i: (0, 0)),
        out_shape=jax.ShapeDtypeStruct((1, 128), jnp.float32),
    )(rows3, lc_rows)


def _fused_kernel(rows_ref, lc_ref, st_ref, out_ref, *, c, fd, kk, dp, n_x, n_xyz, gb):
    c2 = 6 * fd
    st = st_ref[...]
    sum_x, sq_x = st[0, 0], st[0, 1]
    sum_xyz, sq_xyz = st[0, 2], st[0, 3]
    var_x = (sq_x - sum_x * sum_x / n_x) / (n_x - 1)
    var_xyz = (sq_xyz - sum_xyz * sum_xyz / n_xyz) / (n_xyz - 1)
    inv_x = 1.0 / (jnp.sqrt(var_x) + 1e-05)
    inv_xyz = 1.0 / (jnp.sqrt(var_xyz) + 1e-05)

    r3 = rows_ref[...]
    lc = lc_ref[...][:, None, :]
    xyz_n = (r3[..., 0:3] - lc[..., 0:3]) * inv_xyz
    x_n = (r3[..., 3 : 3 + c] - lc[..., 3 : 3 + c]) * inv_x

    li = jax.lax.broadcasted_iota(jnp.int32, (1, 1, c2), 2)
    f = (li % (2 * fd)) // 2
    scale = BETA * jnp.exp(f.astype(jnp.float32) * (-LN_ALPHA / fd))
    phase = jnp.where(li % 2 == 1, HALF_PI, 0.0)

    def bc(a):
        return jnp.broadcast_to(a, (gb, kk, 2 * fd))

    arg = jnp.concatenate(
        [bc(xyz_n[..., 0:1]), bc(xyz_n[..., 1:2]), bc(xyz_n[..., 2:3])], axis=-1
    )
    pe = jnp.sin(arg * scale + phase)
    lcx = jnp.broadcast_to(lc[..., 3 : 3 + c], (gb, kk, c))
    feat = jnp.concatenate([x_n, lcx], axis=-1)
    w = (feat + pe) * pe
    out_ref[...] = jnp.max(w, axis=1) + jnp.sum(w, axis=1) * (1.0 / kk)


def _fused_pe(rows3, lc_rows, stats, c, fd, kk):
    dp = rows3.shape[-1]
    n_rows = lc_rows.shape[0]
    gb = 32 if c < 256 else (16 if c < 512 else 8)
    n_blocks = n_rows // gb
    c2 = 6 * fd
    n_x = n_rows * kk * c
    n_xyz = n_rows * kk * 3
    return pl.pallas_call(
        functools.partial(
            _fused_kernel, c=c, fd=fd, kk=kk, dp=dp, n_x=n_x, n_xyz=n_xyz, gb=gb
        ),
        grid=(n_blocks,),
        in_specs=[
            pl.BlockSpec((gb, kk, dp), lambda i: (i, 0, 0)),
            pl.BlockSpec((gb, dp), lambda i: (i, 0)),
            pl.BlockSpec((1, 128), lambda i: (0, 0)),
        ],
        out_specs=pl.BlockSpec((gb, c2), lambda i: (i, 0)),
        out_shape=jax.ShapeDtypeStruct((n_rows, c2), jnp.float32),
    )(rows3, lc_rows, stats)


def _bn_gelu_kernel(x_ref, w_ref, b_ref, o_ref):
    x = x_ref[...]
    mean = jnp.mean(x, axis=0, keepdims=True)
    var = jnp.mean((x - mean) ** 2, axis=0, keepdims=True)
    xn = (x - mean) / jnp.sqrt(var + 1e-05) * w_ref[...] + b_ref[...]
    o_ref[...] = 0.5 * xn * (1.0 + lax.erf(xn / jnp.sqrt(jnp.float32(2.0))))


def _bn_gelu(pooled, w, b):
    r, c2 = pooled.shape
    return pl.pallas_call(
        _bn_gelu_kernel,
        out_shape=jax.ShapeDtypeStruct((r, c2), jnp.float32),
    )(pooled, w.reshape(1, c2), b.reshape(1, c2))


def _pose_init_kernel(x_ref, o_ref, *, fd, n):
    v = x_ref[0]
    c2 = 6 * fd
    li = jax.lax.broadcasted_iota(jnp.int32, (1, c2), 1)
    f = (li % (2 * fd)) // 2
    scale = BETA * jnp.exp(f.astype(jnp.float32) * (-LN_ALPHA / fd))
    phase = jnp.where(li % 2 == 1, HALF_PI, 0.0)
    arg = jnp.concatenate(
        [
            jnp.broadcast_to(v[:, 0:1], (n, 2 * fd)),
            jnp.broadcast_to(v[:, 1:2], (n, 2 * fd)),
            jnp.broadcast_to(v[:, 2:3], (n, 2 * fd)),
        ],
        axis=-1,
    )
    o_ref[0] = jnp.sin(arg * scale + phase)


def _pose_initial(x_t, fd):
    _, n, _ = x_t.shape
    c2 = 6 * fd
    return pl.pallas_call(
        functools.partial(_pose_init_kernel, fd=fd, n=n),
        grid=(B,),
        in_specs=[pl.BlockSpec((1, n, 3), lambda b: (b, 0, 0))],
        out_specs=pl.BlockSpec((1, n, c2), lambda b: (b, 0, 0)),
        out_shape=jax.ShapeDtypeStruct((B, n, c2), jnp.float32),
    )(x_t)


def _final_kernel(x_ref, o_ref, *, g):
    r = x_ref[...]
    o_ref[...] = jnp.max(r, axis=1) + jnp.sum(r, axis=1) * (1.0 / g)


def _final_reduce(x):
    _, g, c2 = x.shape
    return pl.pallas_call(
        functools.partial(_final_kernel, g=g),
        out_shape=jax.ShapeDtypeStruct((B, c2), jnp.float32),
    )(x)


def kernel(xyz, x, bn_w0, bn_b0, bn_w1, bn_b1, bn_w2, bn_b2, bn_w3, bn_b3):
    bn = [(bn_w0, bn_b0), (bn_w1, bn_b1), (bn_w2, bn_b2), (bn_w3, bn_b3)]
    xfeat = _pose_initial(x.transpose(0, 2, 1), EMBED_DIM // 6)
    cur_xyz = xyz
    for i in range(NUM_STAGES):
        n = N0 >> i
        g = n // 2
        c = xfeat.shape[-1]
        c2 = 2 * c
        fd = c2 // 6
        dp = ((3 + c + 15) // 16) * 16
        fps_idx = _fps_flat(cur_xyz.reshape(B * n, 3).T, g)
        tbl = jnp.concatenate(
            [cur_xyz, xfeat, jnp.zeros((B, n, dp - 3 - c), jnp.float32)], axis=-1
        ).reshape(B * n, dp)
        lc_idx = jnp.minimum(
            fps_idx[None, :] + (jnp.arange(B, dtype=jnp.int32) * n)[:, None],
            B * n - 1,
        ).reshape(-1)
        lc_rows = _sc_gather_rows(tbl, lc_idx)
        knn_idx = _topk(
            lc_rows.reshape(B, g, dp), cur_xyz.transpose(0, 2, 1), n, g, K_NEIGHBORS
        )[..., :K_NEIGHBORS]
        flat_idx = (
            knn_idx + (jnp.arange(B, dtype=jnp.int32) * n)[:, None, None]
        ).reshape(-1)
        rows3 = _sc_gather_rows(tbl, flat_idx).reshape(B * g, K_NEIGHBORS, dp)
        st = _stats(rows3, lc_rows, c, K_NEIGHBORS)
        pooled = _fused_pe(rows3, lc_rows, st, c, fd, K_NEIGHBORS)
        xfeat = _bn_gelu(pooled, bn[i][0], bn[i][1]).reshape(B, g, c2)
        cur_xyz = lc_rows[:, :3].reshape(B, g, 3)
    return _final_reduce(xfeat)
